# Initial kernel scaffold; baseline (speedup 1.0000x reference)
import jax
import jax.numpy as jnp
from jax.experimental import pallas as pl


def kernel(x, edge_index, batch, target, W1, att_src1, att_dst1, b1, W2, att_src2, att_dst2, b2, emb, conv_w, conv_b, fc_xt1_w, fc_xt1_b, fc_g1_w, fc_g1_b, fc1_w, fc1_b, fc2_w, fc2_b, out_w, out_b):
    raise NotImplementedError("write your pallas kernel here")


# speedup vs baseline: 15.5091x; 15.5091x over previous
import functools

import jax
import jax.numpy as jnp
from jax import lax
from jax.experimental import pallas as pl
from jax.experimental.pallas import tpu as pltpu
from jax.experimental.pallas import tpu_sc as plsc

N = 10000
E = 320000
B = 256
D = 128
H = 10
SEQ = 1000
VOCAB = 26

NC = 2
NS = 16
NW = NC * NS
L = 16

K = 100
RB = 1000
NBLK = N // RB
NPAD = 10240
NPT = NPAD // NS
HL = H // NC
EPT1 = E // NS
NRB1 = EPT1 // K
CH1 = 10
NCH1 = NRB1 // CH1
K2 = 100
EPW2 = E // NW
NRB2 = EPW2 // K2
CH2 = 10
NCH2 = NRB2 // CH2
SEGW = B // NW

_GDN = lax.GatherDimensionNumbers(offset_dims=(), collapsed_slice_dims=(0,),
                                  start_index_map=(0,))


def _lane_splat(v, lane):
    idx = jnp.full((16, 1), lane, jnp.int32)
    return lax.gather(v, idx, _GDN, (1,),
                      mode=lax.GatherScatterMode.PROMISE_IN_BOUNDS)


def _tc1_body(x_ref, w1_ref, asp_ref, adp_ref,
              xp_ref, asrc_ref, adst_ref, ms_ref, md_ref):
    i = pl.program_id(0)
    xb = x_ref[...]
    sacc = jnp.zeros((RB, 16), jnp.float32)
    dacc = jnp.zeros((RB, 16), jnp.float32)
    for h in range(H):
        xph = jnp.dot(xb, w1_ref[h], preferred_element_type=jnp.float32)
        xp_ref[h] = xph
        sacc = sacc + jnp.dot(xph, asp_ref[h], preferred_element_type=jnp.float32)
        dacc = dacc + jnp.dot(xph, adp_ref[h], preferred_element_type=jnp.float32)
    asrc_ref[...] = sacc
    adst_ref[...] = dacc
    bs = jnp.max(sacc, axis=0, keepdims=True)
    bd = jnp.max(dacc, axis=0, keepdims=True)

    @pl.when(i == 0)
    def _():
        ms_ref[...] = bs
        md_ref[...] = bd

    @pl.when(i > 0)
    def _():
        ms_ref[...] = jnp.maximum(ms_ref[...], bs)
        md_ref[...] = jnp.maximum(md_ref[...], bd)


def _tc1(x, w1h, asp, adp):
    return pl.pallas_call(
        _tc1_body,
        grid=(NBLK,),
        in_specs=[
            pl.BlockSpec((RB, D), lambda i: (i, 0)),
            pl.BlockSpec((H, D, D), lambda i: (0, 0, 0)),
            pl.BlockSpec((H, D, 16), lambda i: (0, 0, 0)),
            pl.BlockSpec((H, D, 16), lambda i: (0, 0, 0)),
        ],
        out_specs=[
            pl.BlockSpec((H, RB, D), lambda i: (0, i, 0)),
            pl.BlockSpec((RB, 16), lambda i: (i, 0)),
            pl.BlockSpec((RB, 16), lambda i: (i, 0)),
            pl.BlockSpec((1, 16), lambda i: (0, 0)),
            pl.BlockSpec((1, 16), lambda i: (0, 0)),
        ],
        out_shape=[
            jax.ShapeDtypeStruct((H, NPAD, D), jnp.float32),
            jax.ShapeDtypeStruct((NPAD, 16), jnp.float32),
            jax.ShapeDtypeStruct((NPAD, 16), jnp.float32),
            jax.ShapeDtypeStruct((1, 16), jnp.float32),
            jax.ShapeDtypeStruct((1, 16), jnp.float32),
        ],
    )(x, w1h, asp, adp)


def _sc_gat1_body(srcf_ref, dst3_ref, asrc_ref, adst_ref, ms_ref, md_ref,
                  z16_ref, zD_ref, xp_ref,
                  acc1_ref, asum1_ref, w1x_ref,
                  sidxA, didxA, sidxB, didxB,
                  srowsA, drowsA, srowsB, drowsB, wbufA, wbufB, rowsA, rowsB,
                  msl, mdl, accs, asums,
                  smi0, smi1, smi2, smi3, smga1, smga2, smga3,
                  smgb1, smgb2, smgb3, smsa1, smsa2, smsa3,
                  smsb1, smsb2, smsb3):
    cid = lax.axis_index("c")
    sid = lax.axis_index("s")

    pltpu.sync_copy(ms_ref, msl)
    pltpu.sync_copy(md_ref, mdl)
    sh = msl[...] + mdl[...]
    shift = jnp.where(sh > 0.0, sh, 0.2 * sh)

    pltpu.sync_copy(z16_ref.at[pl.ds(sid * NPT, NPT)],
                    asums.at[pl.ds(sid * NPT, NPT)])

    BUF = ((sidxA, didxA, srowsA, drowsA, wbufA, rowsA),
           (sidxB, didxB, srowsB, drowsB, wbufB, rowsB))

    for hl in range(HL):
        h = cid * HL + hl
        hN = h * NPAD
        hsplat = jnp.full((16,), h, jnp.int32)
        pltpu.sync_copy(zD_ref.at[pl.ds(sid * NPT, NPT)],
                        accs.at[pl.ds(sid * NPT, NPT)])
        plsc.subcore_barrier()

        def mk_compute(hl, hsplat):
            def compute(sr, dr, wb, ro):
                if hl == 0:
                    def edge_body(j, c3):
                        rs = sr[j]
                        rd = dr[j]
                        t = rs + rd
                        t = jnp.where(t > 0.0, t, 0.2 * t)
                        w = jnp.exp(t - shift)
                        wb[j] = w
                        wsp = _lane_splat(w, hsplat[0])
                        for f in range(8):
                            ro[j, pl.ds(f * 16, 16)] = (
                                ro[j, pl.ds(f * 16, 16)] * wsp)
                        return c3
                else:
                    def edge_body(j, c3):
                        w = wb[j]
                        wsp = _lane_splat(w, hsplat[0])
                        for f in range(8):
                            ro[j, pl.ds(f * 16, 16)] = (
                                ro[j, pl.ds(f * 16, 16)] * wsp)
                        return c3
                lax.fori_loop(0, K, edge_body, 0)
            return compute

        compute = mk_compute(hl, hsplat)

        def pair_body(t, carry, hl=hl, hN=hN, compute=compute):
            b0 = t * 2
            b1 = b0 + 1
            (sxA, dxA, srA, drA, wbA, roA) = BUF[0]
            (sxB, dxB, srB, drB, wbB, roB) = BUF[1]
            i0 = pltpu.async_copy(srcf_ref.at[sid, b0], sxA, smi0)
            i1 = pltpu.async_copy(dst3_ref.at[sid, b0], dxA, smi1)
            i2 = pltpu.async_copy(srcf_ref.at[sid, b1], sxB, smi2)
            i3 = pltpu.async_copy(dst3_ref.at[sid, b1], dxB, smi3)
            i0.wait()
            i1.wait()
            gA = []
            if hl == 0:
                gA.append(pltpu.async_copy(asrc_ref.at[sxA], srA, smga1))
                gA.append(pltpu.async_copy(adst_ref.at[dxA], drA, smga2))
            else:
                wrow = (sid * NRB1 + b0) * K
                gA.append(pltpu.async_copy(w1x_ref.at[pl.ds(wrow, K)], wbA, smga1))
            gA.append(pltpu.async_copy(
                xp_ref.at[pl.ds(hN, NPAD)].at[sxA], roA, smga3))
            i2.wait()
            i3.wait()
            gB = []
            if hl == 0:
                gB.append(pltpu.async_copy(asrc_ref.at[sxB], srB, smgb1))
                gB.append(pltpu.async_copy(adst_ref.at[dxB], drB, smgb2))
            else:
                wrow = (sid * NRB1 + b1) * K
                gB.append(pltpu.async_copy(w1x_ref.at[pl.ds(wrow, K)], wbB, smgb1))
            gB.append(pltpu.async_copy(
                xp_ref.at[pl.ds(hN, NPAD)].at[sxB], roB, smgb3))
            for g in gA:
                g.wait()
            compute(srA, drA, wbA, roA)
            sA = []
            if hl == 0:
                sA.append(pltpu.async_copy(wbA, asums.at[dxA], smsa1, add=True))
                wrow = (sid * NRB1 + b0) * K
                sA.append(pltpu.async_copy(wbA, w1x_ref.at[pl.ds(wrow, K)], smsa2))
            sA.append(pltpu.async_copy(roA, accs.at[dxA], smsa3, add=True))
            for g in gB:
                g.wait()
            compute(srB, drB, wbB, roB)
            sB = []
            if hl == 0:
                sB.append(pltpu.async_copy(wbB, asums.at[dxB], smsb1, add=True))
                wrow = (sid * NRB1 + b1) * K
                sB.append(pltpu.async_copy(wbB, w1x_ref.at[pl.ds(wrow, K)], smsb2))
            sB.append(pltpu.async_copy(roB, accs.at[dxB], smsb3, add=True))
            for s in sA:
                s.wait()
            for s in sB:
                s.wait()
            return carry

        lax.fori_loop(0, NRB1 // 2, pair_body, 0)
        plsc.subcore_barrier()
        pltpu.sync_copy(accs.at[pl.ds(sid * NPT, NPT)],
                        acc1_ref.at[pl.ds(hN + sid * NPT, NPT)])

    @pl.when(cid == 0)
    def _():
        pltpu.sync_copy(asums.at[pl.ds(sid * NPT, NPT)],
                        asum1_ref.at[pl.ds(sid * NPT, NPT)])


def _sc_gat1(srcf, dst3, asrc, adst, ms, md, z16, zD, xpflat):
    mesh = plsc.VectorSubcoreMesh(core_axis_name="c", subcore_axis_name="s",
                                  num_cores=NC, num_subcores=NS)
    f = pl.kernel(
        _sc_gat1_body,
        out_type=[
            jax.ShapeDtypeStruct((H * NPAD, D), jnp.float32),
            jax.ShapeDtypeStruct((NPAD, 16), jnp.float32),
            jax.ShapeDtypeStruct((E, 16), jnp.float32),
        ],
        mesh=mesh,
        compiler_params=pltpu.CompilerParams(use_tc_tiling_on_sc=False),
        scratch_types=[
            pltpu.VMEM((K,), jnp.int32),
            pltpu.VMEM((K,), jnp.int32),
            pltpu.VMEM((K,), jnp.int32),
            pltpu.VMEM((K,), jnp.int32),
            pltpu.VMEM((K, 16), jnp.float32),
            pltpu.VMEM((K, 16), jnp.float32),
            pltpu.VMEM((K, 16), jnp.float32),
            pltpu.VMEM((K, 16), jnp.float32),
            pltpu.VMEM((K, 16), jnp.float32),
            pltpu.VMEM((K, 16), jnp.float32),
            pltpu.VMEM((K, D), jnp.float32),
            pltpu.VMEM((K, D), jnp.float32),
            pltpu.VMEM((16,), jnp.float32),
            pltpu.VMEM((16,), jnp.float32),
            pltpu.VMEM_SHARED((NPAD, D), jnp.float32),
            pltpu.VMEM_SHARED((NPAD, 16), jnp.float32),
        ] + [pltpu.SemaphoreType.DMA] * 16,
    )
    return f(srcf, dst3, asrc, adst, ms, md, z16, zD, xpflat)


def _tc2_body(acc1_ref, asum_ref, asrc_ref, adst_ref, xp_ref, ms_ref, md_ref,
              batch_ref, b1_ref, w2_ref, a2sp_ref, a2dp_ref,
              h2p_ref, a2s_ref, a2d_ref, ms2_ref, md2_ref, st_ref, en_ref):
    i = pl.program_id(0)
    sh = ms_ref[...] + md_ref[...]
    shift = jnp.where(sh > 0.0, sh, 0.2 * sh)
    t = asrc_ref[...] + adst_ref[...]
    t = jnp.where(t > 0.0, t, 0.2 * t)
    wself = jnp.exp(t - shift)
    inv = 1.0 / (asum_ref[...] + wself + 1e-16)
    h2 = jnp.zeros((RB, D), jnp.float32)
    for h in range(H):
        numer = acc1_ref[h] + wself[:, h:h + 1] * xp_ref[h]
        v = numer * inv[:, h:h + 1] + b1_ref[h][None, :]
        v = jnp.where(v > 0.0, v, jnp.exp(jnp.minimum(v, 0.0)) - 1.0)
        h2 = h2 + jnp.dot(v, w2_ref[h], preferred_element_type=jnp.float32)
    h2p_ref[...] = h2
    a2s = jnp.dot(h2, a2sp_ref[...], preferred_element_type=jnp.float32)
    a2d = jnp.dot(h2, a2dp_ref[...], preferred_element_type=jnp.float32)
    a2s_ref[...] = a2s
    a2d_ref[...] = a2d
    bs = jnp.max(a2s, axis=0, keepdims=True)
    bd = jnp.max(a2d, axis=0, keepdims=True)
    bv = batch_ref[...]
    io = lax.broadcasted_iota(jnp.int32, (1, B), 1)
    lt = jnp.sum((bv < io).astype(jnp.int32), axis=0, keepdims=True)
    le = jnp.sum((bv <= io).astype(jnp.int32), axis=0, keepdims=True)

    @pl.when(i == 0)
    def _():
        ms2_ref[...] = bs
        md2_ref[...] = bd
        st_ref[...] = lt
        en_ref[...] = le

    @pl.when(i > 0)
    def _():
        ms2_ref[...] = jnp.maximum(ms2_ref[...], bs)
        md2_ref[...] = jnp.maximum(md2_ref[...], bd)
        st_ref[...] = st_ref[...] + lt
        en_ref[...] = en_ref[...] + le


def _tc2(acc1, asum1, asrc, adst, xp, ms, md, batch2, b1h, w2h, a2sp, a2dp):
    return pl.pallas_call(
        _tc2_body,
        grid=(NBLK,),
        in_specs=[
            pl.BlockSpec((H, RB, D), lambda i: (0, i, 0)),
            pl.BlockSpec((RB, 16), lambda i: (i, 0)),
            pl.BlockSpec((RB, 16), lambda i: (i, 0)),
            pl.BlockSpec((RB, 16), lambda i: (i, 0)),
            pl.BlockSpec((H, RB, D), lambda i: (0, i, 0)),
            pl.BlockSpec((1, 16), lambda i: (0, 0)),
            pl.BlockSpec((1, 16), lambda i: (0, 0)),
            pl.BlockSpec((RB, 1), lambda i: (i, 0)),
            pl.BlockSpec((H, D), lambda i: (0, 0)),
            pl.BlockSpec((H, D, D), lambda i: (0, 0, 0)),
            pl.BlockSpec((D, 16), lambda i: (0, 0)),
            pl.BlockSpec((D, 16), lambda i: (0, 0)),
        ],
        out_specs=[
            pl.BlockSpec((RB, D), lambda i: (i, 0)),
            pl.BlockSpec((RB, 16), lambda i: (i, 0)),
            pl.BlockSpec((RB, 16), lambda i: (i, 0)),
            pl.BlockSpec((1, 16), lambda i: (0, 0)),
            pl.BlockSpec((1, 16), lambda i: (0, 0)),
            pl.BlockSpec((1, B), lambda i: (0, 0)),
            pl.BlockSpec((1, B), lambda i: (0, 0)),
        ],
        out_shape=[
            jax.ShapeDtypeStruct((NPAD, D), jnp.float32),
            jax.ShapeDtypeStruct((NPAD, 16), jnp.float32),
            jax.ShapeDtypeStruct((NPAD, 16), jnp.float32),
            jax.ShapeDtypeStruct((1, 16), jnp.float32),
            jax.ShapeDtypeStruct((1, 16), jnp.float32),
            jax.ShapeDtypeStruct((1, B), jnp.int32),
            jax.ShapeDtypeStruct((1, B), jnp.int32),
        ],
    )(acc1, asum1, asrc, adst, xp, ms, md, batch2, b1h, w2h, a2sp, a2dp)


def _sc_gat2_body(srcf_ref, dst3_ref, a2s_ref, a2d_ref, ms_ref, md_ref,
                  z16_ref, zD_ref, h2p_ref,
                  acc2_ref, asum2_ref,
                  sidxA, didxA, sidxB, didxB,
                  srowsA, drowsA, srowsB, drowsB, wbufA, wbufB, rowsA, rowsB,
                  msl, mdl, accs, asums,
                  smi0, smi1, smi2, smi3, smga1, smga2, smga3,
                  smgb1, smgb2, smgb3, smsa1, smsa2, smsb1, smsb2):
    cid = lax.axis_index("c")
    sid = lax.axis_index("s")
    wid = sid * NC + cid

    pltpu.sync_copy(ms_ref, msl)
    pltpu.sync_copy(md_ref, mdl)
    sh = msl[...] + mdl[...]
    shift = jnp.where(sh > 0.0, sh, 0.2 * sh)

    pltpu.sync_copy(z16_ref.at[pl.ds(sid * NPT, NPT)],
                    asums.at[pl.ds(sid * NPT, NPT)])
    pltpu.sync_copy(zD_ref.at[pl.ds(sid * NPT, NPT)],
                    accs.at[pl.ds(sid * NPT, NPT)])
    plsc.subcore_barrier()

    def compute(sr, dr, wb, ro):
        def edge_body(j, c3):
            rs = sr[j]
            rd = dr[j]
            t = rs + rd
            t = jnp.where(t > 0.0, t, 0.2 * t)
            w = jnp.exp(t - shift)
            wb[j] = w
            wsp = _lane_splat(w, 0)
            for f in range(8):
                ro[j, pl.ds(f * 16, 16)] = ro[j, pl.ds(f * 16, 16)] * wsp
            return c3
        lax.fori_loop(0, K2, edge_body, 0)

    def pair_body(t, carry):
        b0 = t * 2
        b1 = b0 + 1
        i0 = pltpu.async_copy(srcf_ref.at[wid, b0], sidxA, smi0)
        i1 = pltpu.async_copy(dst3_ref.at[wid, b0], didxA, smi1)
        i2 = pltpu.async_copy(srcf_ref.at[wid, b1], sidxB, smi2)
        i3 = pltpu.async_copy(dst3_ref.at[wid, b1], didxB, smi3)
        i0.wait()
        i1.wait()
        gA = [pltpu.async_copy(a2s_ref.at[sidxA], srowsA, smga1),
              pltpu.async_copy(a2d_ref.at[didxA], drowsA, smga2),
              pltpu.async_copy(h2p_ref.at[sidxA], rowsA, smga3)]
        i2.wait()
        i3.wait()
        gB = [pltpu.async_copy(a2s_ref.at[sidxB], srowsB, smgb1),
              pltpu.async_copy(a2d_ref.at[didxB], drowsB, smgb2),
              pltpu.async_copy(h2p_ref.at[sidxB], rowsB, smgb3)]
        for g in gA:
            g.wait()
        compute(srowsA, drowsA, wbufA, rowsA)
        sA = [pltpu.async_copy(wbufA, asums.at[didxA], smsa1, add=True),
              pltpu.async_copy(rowsA, accs.at[didxA], smsa2, add=True)]
        for g in gB:
            g.wait()
        compute(srowsB, drowsB, wbufB, rowsB)
        sB = [pltpu.async_copy(wbufB, asums.at[didxB], smsb1, add=True),
              pltpu.async_copy(rowsB, accs.at[didxB], smsb2, add=True)]
        for s in sA:
            s.wait()
        for s in sB:
            s.wait()
        return carry

    lax.fori_loop(0, NRB2 // 2, pair_body, 0)
    plsc.subcore_barrier()
    pltpu.sync_copy(accs.at[pl.ds(sid * NPT, NPT)],
                    acc2_ref.at[pl.ds(cid * NPAD + sid * NPT, NPT)])
    pltpu.sync_copy(asums.at[pl.ds(sid * NPT, NPT)],
                    asum2_ref.at[pl.ds(cid * NPAD + sid * NPT, NPT)])


def _sc_gat2(srcf, dst3, a2s, a2d, ms2, md2, z16, zD, h2p):
    mesh = plsc.VectorSubcoreMesh(core_axis_name="c", subcore_axis_name="s",
                                  num_cores=NC, num_subcores=NS)
    f = pl.kernel(
        _sc_gat2_body,
        out_type=[
            jax.ShapeDtypeStruct((NC * NPAD, D), jnp.float32),
            jax.ShapeDtypeStruct((NC * NPAD, 16), jnp.float32),
        ],
        mesh=mesh,
        compiler_params=pltpu.CompilerParams(use_tc_tiling_on_sc=False),
        scratch_types=[
            pltpu.VMEM((K2,), jnp.int32),
            pltpu.VMEM((K2,), jnp.int32),
            pltpu.VMEM((K2,), jnp.int32),
            pltpu.VMEM((K2,), jnp.int32),
            pltpu.VMEM((K2, 16), jnp.float32),
            pltpu.VMEM((K2, 16), jnp.float32),
            pltpu.VMEM((K2, 16), jnp.float32),
            pltpu.VMEM((K2, 16), jnp.float32),
            pltpu.VMEM((K2, 16), jnp.float32),
            pltpu.VMEM((K2, 16), jnp.float32),
            pltpu.VMEM((K2, D), jnp.float32),
            pltpu.VMEM((K2, D), jnp.float32),
            pltpu.VMEM((16,), jnp.float32),
            pltpu.VMEM((16,), jnp.float32),
            pltpu.VMEM_SHARED((NPAD, D), jnp.float32),
            pltpu.VMEM_SHARED((NPAD, 16), jnp.float32),
        ] + [pltpu.SemaphoreType.DMA] * 14,
    )
    return f(srcf, dst3, a2s, a2d, ms2, md2, z16, zD, h2p)


def _tc3_body(a2p_ref, s2p_ref, a2s_ref, a2d_ref, h2p_ref, ms2_ref, md2_ref,
              b2_ref, h2_ref):
    sh = ms2_ref[...] + md2_ref[...]
    shift = jnp.where(sh > 0.0, sh, 0.2 * sh)
    t = a2s_ref[...] + a2d_ref[...]
    t = jnp.where(t > 0.0, t, 0.2 * t)
    wself = jnp.exp(t - shift)[:, 0:1]
    stot = s2p_ref[0][:, 0:1] + s2p_ref[1][:, 0:1] + wself + 1e-16
    numer = a2p_ref[0] + a2p_ref[1] + wself * h2p_ref[...]
    h2_ref[...] = jnp.maximum(numer / stot + b2_ref[...], 0.0)


def _tc3(acc2p, asum2p, a2s, a2d, h2p, ms2, md2, b2row):
    return pl.pallas_call(
        _tc3_body,
        grid=(NBLK,),
        in_specs=[
            pl.BlockSpec((2, RB, D), lambda i: (0, i, 0)),
            pl.BlockSpec((2, RB, 16), lambda i: (0, i, 0)),
            pl.BlockSpec((RB, 16), lambda i: (i, 0)),
            pl.BlockSpec((RB, 16), lambda i: (i, 0)),
            pl.BlockSpec((RB, D), lambda i: (i, 0)),
            pl.BlockSpec((1, 16), lambda i: (0, 0)),
            pl.BlockSpec((1, 16), lambda i: (0, 0)),
            pl.BlockSpec((1, D), lambda i: (0, 0)),
        ],
        out_specs=[pl.BlockSpec((RB, D), lambda i: (i, 0))],
        out_shape=[jax.ShapeDtypeStruct((NPAD, D), jnp.float32)],
    )(acc2p, asum2p, a2s, a2d, h2p, ms2, md2, b2row)[0]


def _sc_pool_body(h2f_ref, st_ref, en_ref, g_ref,
                  bsv, bev, rowv, gbuf, sem):
    cid = lax.axis_index("c")
    sid = lax.axis_index("s")
    wid = sid * NC + cid

    pltpu.sync_copy(st_ref, bsv)
    pltpu.sync_copy(en_ref, bev)
    iot = lax.iota(jnp.int32, 16)
    woff = pl.multiple_of(wid * SEGW, 8)
    wins = bsv[pl.ds(woff, 16)]
    wine = bev[pl.ds(woff, 16)]

    for j in range(SEGW):
        s = wins[j]
        e = wine[j]
        init = tuple(jnp.full((16,), -3.0e38, jnp.float32) for _ in range(8))

        def row_body(r, acc):
            off = pl.multiple_of(r * D, D)
            pltpu.sync_copy(h2f_ref.at[pl.ds(off, D)], rowv)
            return tuple(jnp.maximum(acc[f], rowv[pl.ds(f * 16, 16)])
                         for f in range(8))

        acc = lax.fori_loop(s, e, row_body, init)
        for f in range(8):
            gbuf[pl.ds(j * D + f * 16, 16)] = acc[f]
    pltpu.sync_copy(gbuf, g_ref.at[pl.ds(wid * SEGW * D, SEGW * D)])


def _sc_pool(h2flat, stpad, enpad):
    mesh = plsc.VectorSubcoreMesh(core_axis_name="c", subcore_axis_name="s",
                                  num_cores=NC, num_subcores=NS)
    f = pl.kernel(
        _sc_pool_body,
        out_type=[jax.ShapeDtypeStruct((B * D,), jnp.float32)],
        mesh=mesh,
        compiler_params=pltpu.CompilerParams(use_tc_tiling_on_sc=False),
        scratch_types=[
            pltpu.VMEM((B + 16,), jnp.int32),
            pltpu.VMEM((B + 16,), jnp.int32),
            pltpu.VMEM((D,), jnp.float32),
            pltpu.VMEM((SEGW * D,), jnp.float32),
            pltpu.SemaphoreType.DMA,
        ],
    )
    return f(h2flat, stpad, enpad)[0]


def _tc4_body(t3_ref, emb_ref, wr2_ref, cb_ref, fcw_ref, fcb_ref, xt_ref):
    tcol = t3_ref[0]
    oh = (tcol == lax.broadcasted_iota(jnp.int32, (1, 32), 1))
    et = jnp.dot(oh.astype(jnp.float32), emb_ref[...],
                 preferred_element_type=jnp.float32)
    p = jnp.dot(wr2_ref[...], et, preferred_element_type=jnp.float32)
    c = jnp.zeros((32, 121), jnp.float32)
    for k in range(8):
        c = c + p[k * 32:(k + 1) * 32, k:k + 121]
    c = jnp.maximum(c + cb_ref[...], 0.0)
    acc = lax.dot_general(c, fcw_ref[...],
                          ((( 1,), (1,)), ((0,), (0,))),
                          preferred_element_type=jnp.float32)
    xt_ref[0] = jnp.sum(acc, axis=0, keepdims=True) + fcb_ref[...]


def _tc4(t3, emb32, wr2, cb2, fcw3, fcb):
    return pl.pallas_call(
        _tc4_body,
        grid=(B,),
        in_specs=[
            pl.BlockSpec((1, SEQ, 1), lambda b: (b, 0, 0)),
            pl.BlockSpec((32, D), lambda b: (0, 0)),
            pl.BlockSpec((B, SEQ), lambda b: (0, 0)),
            pl.BlockSpec((32, 1), lambda b: (0, 0)),
            pl.BlockSpec((32, 121, D), lambda b: (0, 0, 0)),
            pl.BlockSpec((1, D), lambda b: (0, 0)),
        ],
        out_specs=[pl.BlockSpec((1, 1, D), lambda b: (b, 0, 0))],
        out_shape=[jax.ShapeDtypeStruct((B, 1, D), jnp.float32)],
    )(t3, emb32, wr2, cb2, fcw3, fcb)[0]


def _tc5_body(g_ref, xt_ref, fg_ref, bg_ref, f1a_ref, f1b_ref, b1_ref,
              f2_ref, b2_ref, ow_ref, ob_ref, out_ref):
    g = jnp.maximum(g_ref[...], 0.0)
    g2 = jnp.maximum(
        jnp.dot(g, fg_ref[...], preferred_element_type=jnp.float32)
        + bg_ref[...], 0.0)
    x1 = jnp.maximum(
        jnp.dot(g2, f1a_ref[...], preferred_element_type=jnp.float32)
        + jnp.dot(xt_ref[...], f1b_ref[...], preferred_element_type=jnp.float32)
        + b1_ref[...], 0.0)
    x2 = jnp.maximum(
        jnp.dot(x1, f2_ref[...], preferred_element_type=jnp.float32)
        + b2_ref[...], 0.0)
    out_ref[...] = (jnp.dot(x2, ow_ref[...], preferred_element_type=jnp.float32)
                    + ob_ref[...])


def _tc5(g, xt, fg, bg, f1a, f1b, b1f, f2, b2f, ow, ob):
    return pl.pallas_call(
        _tc5_body,
        out_shape=jax.ShapeDtypeStruct((B, 1), jnp.float32),
    )(g, xt, fg, bg, f1a, f1b, b1f, f2, b2f, ow, ob)


def kernel(x, edge_index, batch, target, W1, att_src1, att_dst1, b1,
           W2, att_src2, att_dst2, b2, emb, conv_w, conv_b,
           fc_xt1_w, fc_xt1_b, fc_g1_w, fc_g1_b, fc1_w, fc1_b,
           fc2_w, fc2_b, out_w, out_b):
    srcf1 = edge_index[0].reshape(NS, NRB1, K)
    dst31 = edge_index[1].reshape(NS, NRB1, K)
    srcf2 = edge_index[0].reshape(NW, NRB2, K2)
    dst32 = edge_index[1].reshape(NW, NRB2, K2)
    w1h = W1.reshape(D, H, D).transpose(1, 0, 2)
    eye16 = jnp.eye(16, dtype=jnp.float32)[:H]
    asp = att_src1[:, :, None] * eye16[:, None, :]
    adp = att_dst1[:, :, None] * eye16[:, None, :]
    z16 = jnp.zeros((NPAD, 16), jnp.float32)
    zD = jnp.zeros((NPAD, D), jnp.float32)
    b1h = b1.reshape(H, D)
    w2h = W2.reshape(H, D, D)
    a2sp = att_src2[0][:, None] * jnp.eye(16, dtype=jnp.float32)[0][None, :]
    a2dp = att_dst2[0][:, None] * jnp.eye(16, dtype=jnp.float32)[0][None, :]
    batch2 = batch.reshape(N, 1)
    b2row = b2.reshape(1, D)

    xp, asrc, adst, ms1, md1 = _tc1(x, w1h, asp, adp)
    acc1, asum1, _w1x = _sc_gat1(srcf1, dst31, asrc, adst,
                           ms1.reshape(16), md1.reshape(16),
                           z16, zD, xp.reshape(H * NPAD, D))
    acc1 = acc1.reshape(H, NPAD, D)

    h2p, a2s, a2d, ms2, md2, st, en = _tc2(
        acc1, asum1, asrc, adst, xp, ms1, md1, batch2, b1h, w2h, a2sp, a2dp)

    acc2, asum2 = _sc_gat2(srcf2, dst32, a2s, a2d,
                           ms2.reshape(16), md2.reshape(16), z16, zD, h2p)

    h2 = _tc3(acc2.reshape(2, NPAD, D), asum2.reshape(2, NPAD, 16),
              a2s, a2d, h2p, ms2, md2, b2row)

    stpad = jnp.pad(st.reshape(B), (0, 16))
    enpad = jnp.pad(en.reshape(B), (0, 16))
    g = _sc_pool(h2.reshape(NPAD * D), stpad, enpad).reshape(B, D)

    t3 = target[:, :, None]
    emb32 = jnp.pad(emb, ((0, 32 - VOCAB), (0, 0)))
    wr2 = conv_w.transpose(2, 0, 1).reshape(B, SEQ)
    cb2 = conv_b.reshape(32, 1)
    fcw3 = fc_xt1_w.reshape(32, 121, D)
    xt = _tc4(t3, emb32, wr2, cb2, fcw3, fc_xt1_b.reshape(1, D)).reshape(B, D)

    return _tc5(g, xt, fc_g1_w, fc_g1_b.reshape(1, D),
                fc1_w[:D], fc1_w[D:], fc1_b.reshape(1, 1024),
                fc2_w, fc2_b.reshape(1, B), out_w, out_b.reshape(1, 1))


# speedup vs baseline: 18.1351x; 1.1693x over previous
import functools

import jax
import jax.numpy as jnp
from jax import lax
from jax.experimental import pallas as pl
from jax.experimental.pallas import tpu as pltpu
from jax.experimental.pallas import tpu_sc as plsc

N = 10000
E = 320000
B = 256
D = 128
H = 10
SEQ = 1000
VOCAB = 26

NC = 2
NS = 16
NW = NC * NS
L = 16

K = 100
RB = 1000
NBLK = N // RB
NPAD = 10240
NPT = NPAD // NS
HL = H // NC
EPT1 = E // NS
NRB1 = EPT1 // K
CH1 = 10
NCH1 = NRB1 // CH1
K2 = 100
EPW2 = E // NW
NRB2 = EPW2 // K2
CH2 = 10
NCH2 = NRB2 // CH2
SEGW = B // NW

_GDN = lax.GatherDimensionNumbers(offset_dims=(), collapsed_slice_dims=(0,),
                                  start_index_map=(0,))


def _lane_splat(v, lane):
    idx = jnp.full((16, 1), lane, jnp.int32)
    return lax.gather(v, idx, _GDN, (1,),
                      mode=lax.GatherScatterMode.PROMISE_IN_BOUNDS)


def _tc1_body(x_ref, w1_ref, asp_ref, adp_ref,
              xp_ref, asrc_ref, adst_ref, ms_ref, md_ref):
    i = pl.program_id(0)
    xb = x_ref[...]
    sacc = jnp.zeros((RB, 16), jnp.float32)
    dacc = jnp.zeros((RB, 16), jnp.float32)
    for h in range(H):
        xph = jnp.dot(xb, w1_ref[h], preferred_element_type=jnp.float32)
        xp_ref[h] = xph
        sacc = sacc + jnp.dot(xph, asp_ref[h], preferred_element_type=jnp.float32)
        dacc = dacc + jnp.dot(xph, adp_ref[h], preferred_element_type=jnp.float32)
    asrc_ref[...] = sacc
    adst_ref[...] = dacc
    bs = jnp.max(sacc, axis=0, keepdims=True)
    bd = jnp.max(dacc, axis=0, keepdims=True)

    @pl.when(i == 0)
    def _():
        ms_ref[...] = bs
        md_ref[...] = bd

    @pl.when(i > 0)
    def _():
        ms_ref[...] = jnp.maximum(ms_ref[...], bs)
        md_ref[...] = jnp.maximum(md_ref[...], bd)


def _tc1(x, w1h, asp, adp):
    return pl.pallas_call(
        _tc1_body,
        grid=(NBLK,),
        in_specs=[
            pl.BlockSpec((RB, D), lambda i: (i, 0)),
            pl.BlockSpec((H, D, D), lambda i: (0, 0, 0)),
            pl.BlockSpec((H, D, 16), lambda i: (0, 0, 0)),
            pl.BlockSpec((H, D, 16), lambda i: (0, 0, 0)),
        ],
        out_specs=[
            pl.BlockSpec((H, RB, D), lambda i: (0, i, 0)),
            pl.BlockSpec((RB, 16), lambda i: (i, 0)),
            pl.BlockSpec((RB, 16), lambda i: (i, 0)),
            pl.BlockSpec((1, 16), lambda i: (0, 0)),
            pl.BlockSpec((1, 16), lambda i: (0, 0)),
        ],
        out_shape=[
            jax.ShapeDtypeStruct((H, NPAD, D), jnp.float32),
            jax.ShapeDtypeStruct((NPAD, 16), jnp.float32),
            jax.ShapeDtypeStruct((NPAD, 16), jnp.float32),
            jax.ShapeDtypeStruct((1, 16), jnp.float32),
            jax.ShapeDtypeStruct((1, 16), jnp.float32),
        ],
    )(x, w1h, asp, adp)


def _sc_gat1_body(srcf_ref, dst3_ref, asrc_ref, adst_ref, ms_ref, md_ref,
                  z16_ref, zD_ref, xp_ref,
                  acc1_ref, asum1_ref, w1x_ref,
                  sidxA, didxA, sidxB, didxB,
                  srowsA, drowsA, srowsB, drowsB, wbufA, wbufB, rowsA, rowsB,
                  msl, mdl, accs, asums,
                  smi0, smi1, smi2, smi3, smga1, smga2, smga3,
                  smgb1, smgb2, smgb3, smsa1, smsa2, smsa3,
                  smsb1, smsb2, smsb3):
    cid = lax.axis_index("c")
    sid = lax.axis_index("s")

    pltpu.sync_copy(ms_ref, msl)
    pltpu.sync_copy(md_ref, mdl)
    sh = msl[...] + mdl[...]
    shift = jnp.where(sh > 0.0, sh, 0.2 * sh)

    pltpu.sync_copy(z16_ref.at[pl.ds(sid * NPT, NPT)],
                    asums.at[pl.ds(sid * NPT, NPT)])

    BUF = ((sidxA, didxA, srowsA, drowsA, wbufA, rowsA),
           (sidxB, didxB, srowsB, drowsB, wbufB, rowsB))

    for hl in range(HL):
        h = cid * HL + hl
        hN = h * NPAD
        hsplat = jnp.full((16,), h, jnp.int32)
        pltpu.sync_copy(zD_ref.at[pl.ds(sid * NPT, NPT)],
                        accs.at[pl.ds(sid * NPT, NPT)])
        plsc.subcore_barrier()

        def mk_compute(hl, hsplat):
            def compute(sr, dr, wb, ro):
                if hl == 0:
                    def edge_body(t2, c3):
                        j = t2 * 2
                        rs = sr[j]
                        rd = dr[j]
                        rs2 = sr[j + 1]
                        rd2 = dr[j + 1]
                        t = rs + rd
                        t2v = rs2 + rd2
                        t = jnp.where(t > 0.0, t, 0.2 * t)
                        t2v = jnp.where(t2v > 0.0, t2v, 0.2 * t2v)
                        w = jnp.exp(t - shift)
                        w2 = jnp.exp(t2v - shift)
                        wb[j] = w
                        wb[j + 1] = w2
                        wsp = _lane_splat(w, hsplat[0])
                        wsp2 = _lane_splat(w2, hsplat[0])
                        for f in range(8):
                            ro[j, pl.ds(f * 16, 16)] = (
                                ro[j, pl.ds(f * 16, 16)] * wsp)
                        for f in range(8):
                            ro[j + 1, pl.ds(f * 16, 16)] = (
                                ro[j + 1, pl.ds(f * 16, 16)] * wsp2)
                        return c3
                else:
                    def edge_body(t2, c3):
                        j = t2 * 2
                        w = wb[j]
                        w2 = wb[j + 1]
                        wsp = _lane_splat(w, hsplat[0])
                        wsp2 = _lane_splat(w2, hsplat[0])
                        for f in range(8):
                            ro[j, pl.ds(f * 16, 16)] = (
                                ro[j, pl.ds(f * 16, 16)] * wsp)
                        for f in range(8):
                            ro[j + 1, pl.ds(f * 16, 16)] = (
                                ro[j + 1, pl.ds(f * 16, 16)] * wsp2)
                        return c3
                lax.fori_loop(0, K // 2, edge_body, 0)
            return compute

        compute = mk_compute(hl, hsplat)

        def pair_body(t, carry, hl=hl, hN=hN, compute=compute):
            b0 = t * 2
            b1 = b0 + 1
            (sxA, dxA, srA, drA, wbA, roA) = BUF[0]
            (sxB, dxB, srB, drB, wbB, roB) = BUF[1]
            i0 = pltpu.async_copy(srcf_ref.at[sid, b0], sxA, smi0)
            i1 = pltpu.async_copy(dst3_ref.at[sid, b0], dxA, smi1)
            i2 = pltpu.async_copy(srcf_ref.at[sid, b1], sxB, smi2)
            i3 = pltpu.async_copy(dst3_ref.at[sid, b1], dxB, smi3)
            i0.wait()
            i1.wait()
            gA = []
            if hl == 0:
                gA.append(pltpu.async_copy(asrc_ref.at[sxA], srA, smga1))
                gA.append(pltpu.async_copy(adst_ref.at[dxA], drA, smga2))
            else:
                wrow = (sid * NRB1 + b0) * K
                gA.append(pltpu.async_copy(w1x_ref.at[pl.ds(wrow, K)], wbA, smga1))
            gA.append(pltpu.async_copy(
                xp_ref.at[pl.ds(hN, NPAD)].at[sxA], roA, smga3))
            i2.wait()
            i3.wait()
            gB = []
            if hl == 0:
                gB.append(pltpu.async_copy(asrc_ref.at[sxB], srB, smgb1))
                gB.append(pltpu.async_copy(adst_ref.at[dxB], drB, smgb2))
            else:
                wrow = (sid * NRB1 + b1) * K
                gB.append(pltpu.async_copy(w1x_ref.at[pl.ds(wrow, K)], wbB, smgb1))
            gB.append(pltpu.async_copy(
                xp_ref.at[pl.ds(hN, NPAD)].at[sxB], roB, smgb3))
            for g in gA:
                g.wait()
            compute(srA, drA, wbA, roA)
            sA = []
            if hl == 0:
                sA.append(pltpu.async_copy(wbA, asums.at[dxA], smsa1, add=True))
                wrow = (sid * NRB1 + b0) * K
                sA.append(pltpu.async_copy(wbA, w1x_ref.at[pl.ds(wrow, K)], smsa2))
            sA.append(pltpu.async_copy(roA, accs.at[dxA], smsa3, add=True))
            for g in gB:
                g.wait()
            compute(srB, drB, wbB, roB)
            sB = []
            if hl == 0:
                sB.append(pltpu.async_copy(wbB, asums.at[dxB], smsb1, add=True))
                wrow = (sid * NRB1 + b1) * K
                sB.append(pltpu.async_copy(wbB, w1x_ref.at[pl.ds(wrow, K)], smsb2))
            sB.append(pltpu.async_copy(roB, accs.at[dxB], smsb3, add=True))
            for s in sA:
                s.wait()
            for s in sB:
                s.wait()
            return carry

        lax.fori_loop(0, NRB1 // 2, pair_body, 0)
        plsc.subcore_barrier()
        pltpu.sync_copy(accs.at[pl.ds(sid * NPT, NPT)],
                        acc1_ref.at[pl.ds(hN + sid * NPT, NPT)])

    @pl.when(cid == 0)
    def _():
        pltpu.sync_copy(asums.at[pl.ds(sid * NPT, NPT)],
                        asum1_ref.at[pl.ds(sid * NPT, NPT)])


def _sc_gat1(srcf, dst3, asrc, adst, ms, md, z16, zD, xpflat):
    mesh = plsc.VectorSubcoreMesh(core_axis_name="c", subcore_axis_name="s",
                                  num_cores=NC, num_subcores=NS)
    f = pl.kernel(
        _sc_gat1_body,
        out_type=[
            jax.ShapeDtypeStruct((H * NPAD, D), jnp.float32),
            jax.ShapeDtypeStruct((NPAD, 16), jnp.float32),
            jax.ShapeDtypeStruct((E, 16), jnp.float32),
        ],
        mesh=mesh,
        compiler_params=pltpu.CompilerParams(use_tc_tiling_on_sc=False),
        scratch_types=[
            pltpu.VMEM((K,), jnp.int32),
            pltpu.VMEM((K,), jnp.int32),
            pltpu.VMEM((K,), jnp.int32),
            pltpu.VMEM((K,), jnp.int32),
            pltpu.VMEM((K, 16), jnp.float32),
            pltpu.VMEM((K, 16), jnp.float32),
            pltpu.VMEM((K, 16), jnp.float32),
            pltpu.VMEM((K, 16), jnp.float32),
            pltpu.VMEM((K, 16), jnp.float32),
            pltpu.VMEM((K, 16), jnp.float32),
            pltpu.VMEM((K, D), jnp.float32),
            pltpu.VMEM((K, D), jnp.float32),
            pltpu.VMEM((16,), jnp.float32),
            pltpu.VMEM((16,), jnp.float32),
            pltpu.VMEM_SHARED((NPAD, D), jnp.float32),
            pltpu.VMEM_SHARED((NPAD, 16), jnp.float32),
        ] + [pltpu.SemaphoreType.DMA] * 16,
    )
    return f(srcf, dst3, asrc, adst, ms, md, z16, zD, xpflat)


def _tc2_body(acc1_ref, asum_ref, asrc_ref, adst_ref, xp_ref, ms_ref, md_ref,
              batch_ref, b1_ref, w2_ref, a2sp_ref, a2dp_ref,
              h2p_ref, a2s_ref, a2d_ref, ms2_ref, md2_ref, st_ref, en_ref):
    i = pl.program_id(0)
    sh = ms_ref[...] + md_ref[...]
    shift = jnp.where(sh > 0.0, sh, 0.2 * sh)
    t = asrc_ref[...] + adst_ref[...]
    t = jnp.where(t > 0.0, t, 0.2 * t)
    wself = jnp.exp(t - shift)
    inv = 1.0 / (asum_ref[...] + wself + 1e-16)
    h2 = jnp.zeros((RB, D), jnp.float32)
    for h in range(H):
        numer = acc1_ref[h] + wself[:, h:h + 1] * xp_ref[h]
        v = numer * inv[:, h:h + 1] + b1_ref[h][None, :]
        v = jnp.where(v > 0.0, v, jnp.exp(jnp.minimum(v, 0.0)) - 1.0)
        h2 = h2 + jnp.dot(v, w2_ref[h], preferred_element_type=jnp.float32)
    h2p_ref[...] = h2
    a2s = jnp.dot(h2, a2sp_ref[...], preferred_element_type=jnp.float32)
    a2d = jnp.dot(h2, a2dp_ref[...], preferred_element_type=jnp.float32)
    a2s_ref[...] = a2s
    a2d_ref[...] = a2d
    bs = jnp.max(a2s, axis=0, keepdims=True)
    bd = jnp.max(a2d, axis=0, keepdims=True)
    bv = batch_ref[...]
    io = lax.broadcasted_iota(jnp.int32, (1, B), 1)
    lt = jnp.sum((bv < io).astype(jnp.int32), axis=0, keepdims=True)
    le = jnp.sum((bv <= io).astype(jnp.int32), axis=0, keepdims=True)

    @pl.when(i == 0)
    def _():
        ms2_ref[...] = bs
        md2_ref[...] = bd
        st_ref[...] = lt
        en_ref[...] = le

    @pl.when(i > 0)
    def _():
        ms2_ref[...] = jnp.maximum(ms2_ref[...], bs)
        md2_ref[...] = jnp.maximum(md2_ref[...], bd)
        st_ref[...] = st_ref[...] + lt
        en_ref[...] = en_ref[...] + le


def _tc2(acc1, asum1, asrc, adst, xp, ms, md, batch2, b1h, w2h, a2sp, a2dp):
    return pl.pallas_call(
        _tc2_body,
        grid=(NBLK,),
        in_specs=[
            pl.BlockSpec((H, RB, D), lambda i: (0, i, 0)),
            pl.BlockSpec((RB, 16), lambda i: (i, 0)),
            pl.BlockSpec((RB, 16), lambda i: (i, 0)),
            pl.BlockSpec((RB, 16), lambda i: (i, 0)),
            pl.BlockSpec((H, RB, D), lambda i: (0, i, 0)),
            pl.BlockSpec((1, 16), lambda i: (0, 0)),
            pl.BlockSpec((1, 16), lambda i: (0, 0)),
            pl.BlockSpec((RB, 1), lambda i: (i, 0)),
            pl.BlockSpec((H, D), lambda i: (0, 0)),
            pl.BlockSpec((H, D, D), lambda i: (0, 0, 0)),
            pl.BlockSpec((D, 16), lambda i: (0, 0)),
            pl.BlockSpec((D, 16), lambda i: (0, 0)),
        ],
        out_specs=[
            pl.BlockSpec((RB, D), lambda i: (i, 0)),
            pl.BlockSpec((RB, 16), lambda i: (i, 0)),
            pl.BlockSpec((RB, 16), lambda i: (i, 0)),
            pl.BlockSpec((1, 16), lambda i: (0, 0)),
            pl.BlockSpec((1, 16), lambda i: (0, 0)),
            pl.BlockSpec((1, B), lambda i: (0, 0)),
            pl.BlockSpec((1, B), lambda i: (0, 0)),
        ],
        out_shape=[
            jax.ShapeDtypeStruct((NPAD, D), jnp.float32),
            jax.ShapeDtypeStruct((NPAD, 16), jnp.float32),
            jax.ShapeDtypeStruct((NPAD, 16), jnp.float32),
            jax.ShapeDtypeStruct((1, 16), jnp.float32),
            jax.ShapeDtypeStruct((1, 16), jnp.float32),
            jax.ShapeDtypeStruct((1, B), jnp.int32),
            jax.ShapeDtypeStruct((1, B), jnp.int32),
        ],
    )(acc1, asum1, asrc, adst, xp, ms, md, batch2, b1h, w2h, a2sp, a2dp)


def _sc_gat2_body(srcf_ref, dst3_ref, a2s_ref, a2d_ref, ms_ref, md_ref,
                  z16_ref, zD_ref, h2p_ref,
                  acc2_ref, asum2_ref,
                  sidxA, didxA, sidxB, didxB,
                  srowsA, drowsA, srowsB, drowsB, wbufA, wbufB, rowsA, rowsB,
                  msl, mdl, accs, asums,
                  smi0, smi1, smi2, smi3, smga1, smga2, smga3,
                  smgb1, smgb2, smgb3, smsa1, smsa2, smsb1, smsb2):
    cid = lax.axis_index("c")
    sid = lax.axis_index("s")
    wid = sid * NC + cid

    pltpu.sync_copy(ms_ref, msl)
    pltpu.sync_copy(md_ref, mdl)
    sh = msl[...] + mdl[...]
    shift = jnp.where(sh > 0.0, sh, 0.2 * sh)

    pltpu.sync_copy(z16_ref.at[pl.ds(sid * NPT, NPT)],
                    asums.at[pl.ds(sid * NPT, NPT)])
    pltpu.sync_copy(zD_ref.at[pl.ds(sid * NPT, NPT)],
                    accs.at[pl.ds(sid * NPT, NPT)])
    plsc.subcore_barrier()

    def compute(sr, dr, wb, ro):
        def edge_body(t2, c3):
            j = t2 * 2
            rs = sr[j]
            rd = dr[j]
            rs2 = sr[j + 1]
            rd2 = dr[j + 1]
            t = rs + rd
            t2v = rs2 + rd2
            t = jnp.where(t > 0.0, t, 0.2 * t)
            t2v = jnp.where(t2v > 0.0, t2v, 0.2 * t2v)
            w = jnp.exp(t - shift)
            w2 = jnp.exp(t2v - shift)
            wb[j] = w
            wb[j + 1] = w2
            wsp = _lane_splat(w, 0)
            wsp2 = _lane_splat(w2, 0)
            for f in range(8):
                ro[j, pl.ds(f * 16, 16)] = ro[j, pl.ds(f * 16, 16)] * wsp
            for f in range(8):
                ro[j + 1, pl.ds(f * 16, 16)] = ro[j + 1, pl.ds(f * 16, 16)] * wsp2
            return c3
        lax.fori_loop(0, K2 // 2, edge_body, 0)

    def pair_body(t, carry):
        b0 = t * 2
        b1 = b0 + 1
        i0 = pltpu.async_copy(srcf_ref.at[wid, b0], sidxA, smi0)
        i1 = pltpu.async_copy(dst3_ref.at[wid, b0], didxA, smi1)
        i2 = pltpu.async_copy(srcf_ref.at[wid, b1], sidxB, smi2)
        i3 = pltpu.async_copy(dst3_ref.at[wid, b1], didxB, smi3)
        i0.wait()
        i1.wait()
        gA = [pltpu.async_copy(a2s_ref.at[sidxA], srowsA, smga1),
              pltpu.async_copy(a2d_ref.at[didxA], drowsA, smga2),
              pltpu.async_copy(h2p_ref.at[sidxA], rowsA, smga3)]
        i2.wait()
        i3.wait()
        gB = [pltpu.async_copy(a2s_ref.at[sidxB], srowsB, smgb1),
              pltpu.async_copy(a2d_ref.at[didxB], drowsB, smgb2),
              pltpu.async_copy(h2p_ref.at[sidxB], rowsB, smgb3)]
        for g in gA:
            g.wait()
        compute(srowsA, drowsA, wbufA, rowsA)
        sA = [pltpu.async_copy(wbufA, asums.at[didxA], smsa1, add=True),
              pltpu.async_copy(rowsA, accs.at[didxA], smsa2, add=True)]
        for g in gB:
            g.wait()
        compute(srowsB, drowsB, wbufB, rowsB)
        sB = [pltpu.async_copy(wbufB, asums.at[didxB], smsb1, add=True),
              pltpu.async_copy(rowsB, accs.at[didxB], smsb2, add=True)]
        for s in sA:
            s.wait()
        for s in sB:
            s.wait()
        return carry

    lax.fori_loop(0, NRB2 // 2, pair_body, 0)
    plsc.subcore_barrier()
    pltpu.sync_copy(accs.at[pl.ds(sid * NPT, NPT)],
                    acc2_ref.at[pl.ds(cid * NPAD + sid * NPT, NPT)])
    pltpu.sync_copy(asums.at[pl.ds(sid * NPT, NPT)],
                    asum2_ref.at[pl.ds(cid * NPAD + sid * NPT, NPT)])


def _sc_gat2(srcf, dst3, a2s, a2d, ms2, md2, z16, zD, h2p):
    mesh = plsc.VectorSubcoreMesh(core_axis_name="c", subcore_axis_name="s",
                                  num_cores=NC, num_subcores=NS)
    f = pl.kernel(
        _sc_gat2_body,
        out_type=[
            jax.ShapeDtypeStruct((NC * NPAD, D), jnp.float32),
            jax.ShapeDtypeStruct((NC * NPAD, 16), jnp.float32),
        ],
        mesh=mesh,
        compiler_params=pltpu.CompilerParams(use_tc_tiling_on_sc=False),
        scratch_types=[
            pltpu.VMEM((K2,), jnp.int32),
            pltpu.VMEM((K2,), jnp.int32),
            pltpu.VMEM((K2,), jnp.int32),
            pltpu.VMEM((K2,), jnp.int32),
            pltpu.VMEM((K2, 16), jnp.float32),
            pltpu.VMEM((K2, 16), jnp.float32),
            pltpu.VMEM((K2, 16), jnp.float32),
            pltpu.VMEM((K2, 16), jnp.float32),
            pltpu.VMEM((K2, 16), jnp.float32),
            pltpu.VMEM((K2, 16), jnp.float32),
            pltpu.VMEM((K2, D), jnp.float32),
            pltpu.VMEM((K2, D), jnp.float32),
            pltpu.VMEM((16,), jnp.float32),
            pltpu.VMEM((16,), jnp.float32),
            pltpu.VMEM_SHARED((NPAD, D), jnp.float32),
            pltpu.VMEM_SHARED((NPAD, 16), jnp.float32),
        ] + [pltpu.SemaphoreType.DMA] * 14,
    )
    return f(srcf, dst3, a2s, a2d, ms2, md2, z16, zD, h2p)


def _tc3_body(a2p_ref, s2p_ref, a2s_ref, a2d_ref, h2p_ref, ms2_ref, md2_ref,
              b2_ref, h2_ref):
    sh = ms2_ref[...] + md2_ref[...]
    shift = jnp.where(sh > 0.0, sh, 0.2 * sh)
    t = a2s_ref[...] + a2d_ref[...]
    t = jnp.where(t > 0.0, t, 0.2 * t)
    wself = jnp.exp(t - shift)[:, 0:1]
    stot = s2p_ref[0][:, 0:1] + s2p_ref[1][:, 0:1] + wself + 1e-16
    numer = a2p_ref[0] + a2p_ref[1] + wself * h2p_ref[...]
    h2_ref[...] = jnp.maximum(numer / stot + b2_ref[...], 0.0)


def _tc3(acc2p, asum2p, a2s, a2d, h2p, ms2, md2, b2row):
    return pl.pallas_call(
        _tc3_body,
        grid=(NBLK,),
        in_specs=[
            pl.BlockSpec((2, RB, D), lambda i: (0, i, 0)),
            pl.BlockSpec((2, RB, 16), lambda i: (0, i, 0)),
            pl.BlockSpec((RB, 16), lambda i: (i, 0)),
            pl.BlockSpec((RB, 16), lambda i: (i, 0)),
            pl.BlockSpec((RB, D), lambda i: (i, 0)),
            pl.BlockSpec((1, 16), lambda i: (0, 0)),
            pl.BlockSpec((1, 16), lambda i: (0, 0)),
            pl.BlockSpec((1, D), lambda i: (0, 0)),
        ],
        out_specs=[pl.BlockSpec((RB, D), lambda i: (i, 0))],
        out_shape=[jax.ShapeDtypeStruct((NPAD, D), jnp.float32)],
    )(acc2p, asum2p, a2s, a2d, h2p, ms2, md2, b2row)[0]


def _sc_pool_body(h2f_ref, st_ref, en_ref, g_ref,
                  bsv, bev, rowv, gbuf, sem):
    cid = lax.axis_index("c")
    sid = lax.axis_index("s")
    wid = sid * NC + cid

    pltpu.sync_copy(st_ref, bsv)
    pltpu.sync_copy(en_ref, bev)
    iot = lax.iota(jnp.int32, 16)
    woff = pl.multiple_of(wid * SEGW, 8)
    wins = bsv[pl.ds(woff, 16)]
    wine = bev[pl.ds(woff, 16)]

    for j in range(SEGW):
        s = wins[j]
        e = wine[j]
        init = tuple(jnp.full((16,), -3.0e38, jnp.float32) for _ in range(8))

        def row_body(r, acc):
            off = pl.multiple_of(r * D, D)
            pltpu.sync_copy(h2f_ref.at[pl.ds(off, D)], rowv)
            return tuple(jnp.maximum(acc[f], rowv[pl.ds(f * 16, 16)])
                         for f in range(8))

        acc = lax.fori_loop(s, e, row_body, init)
        for f in range(8):
            gbuf[pl.ds(j * D + f * 16, 16)] = acc[f]
    pltpu.sync_copy(gbuf, g_ref.at[pl.ds(wid * SEGW * D, SEGW * D)])


def _sc_pool(h2flat, stpad, enpad):
    mesh = plsc.VectorSubcoreMesh(core_axis_name="c", subcore_axis_name="s",
                                  num_cores=NC, num_subcores=NS)
    f = pl.kernel(
        _sc_pool_body,
        out_type=[jax.ShapeDtypeStruct((B * D,), jnp.float32)],
        mesh=mesh,
        compiler_params=pltpu.CompilerParams(use_tc_tiling_on_sc=False),
        scratch_types=[
            pltpu.VMEM((B + 16,), jnp.int32),
            pltpu.VMEM((B + 16,), jnp.int32),
            pltpu.VMEM((D,), jnp.float32),
            pltpu.VMEM((SEGW * D,), jnp.float32),
            pltpu.SemaphoreType.DMA,
        ],
    )
    return f(h2flat, stpad, enpad)[0]


def _tc4_body(t3_ref, emb_ref, wr2_ref, cb_ref, fcw_ref, fcb_ref, xt_ref):
    tcol = t3_ref[0]
    oh = (tcol == lax.broadcasted_iota(jnp.int32, (1, 32), 1))
    et = jnp.dot(oh.astype(jnp.float32), emb_ref[...],
                 preferred_element_type=jnp.float32)
    p = jnp.dot(wr2_ref[...], et, preferred_element_type=jnp.float32)
    c = jnp.zeros((32, 121), jnp.float32)
    for k in range(8):
        c = c + p[k * 32:(k + 1) * 32, k:k + 121]
    c = jnp.maximum(c + cb_ref[...], 0.0)
    acc = lax.dot_general(c, fcw_ref[...],
                          ((( 1,), (1,)), ((0,), (0,))),
                          preferred_element_type=jnp.float32)
    xt_ref[0] = jnp.sum(acc, axis=0, keepdims=True) + fcb_ref[...]


def _tc4(t3, emb32, wr2, cb2, fcw3, fcb):
    return pl.pallas_call(
        _tc4_body,
        grid=(B,),
        in_specs=[
            pl.BlockSpec((1, SEQ, 1), lambda b: (b, 0, 0)),
            pl.BlockSpec((32, D), lambda b: (0, 0)),
            pl.BlockSpec((B, SEQ), lambda b: (0, 0)),
            pl.BlockSpec((32, 1), lambda b: (0, 0)),
            pl.BlockSpec((32, 121, D), lambda b: (0, 0, 0)),
            pl.BlockSpec((1, D), lambda b: (0, 0)),
        ],
        out_specs=[pl.BlockSpec((1, 1, D), lambda b: (b, 0, 0))],
        out_shape=[jax.ShapeDtypeStruct((B, 1, D), jnp.float32)],
    )(t3, emb32, wr2, cb2, fcw3, fcb)[0]


def _tc5_body(g_ref, xt_ref, fg_ref, bg_ref, f1a_ref, f1b_ref, b1_ref,
              f2_ref, b2_ref, ow_ref, ob_ref, out_ref):
    g = jnp.maximum(g_ref[...], 0.0)
    g2 = jnp.maximum(
        jnp.dot(g, fg_ref[...], preferred_element_type=jnp.float32)
        + bg_ref[...], 0.0)
    x1 = jnp.maximum(
        jnp.dot(g2, f1a_ref[...], preferred_element_type=jnp.float32)
        + jnp.dot(xt_ref[...], f1b_ref[...], preferred_element_type=jnp.float32)
        + b1_ref[...], 0.0)
    x2 = jnp.maximum(
        jnp.dot(x1, f2_ref[...], preferred_element_type=jnp.float32)
        + b2_ref[...], 0.0)
    out_ref[...] = (jnp.dot(x2, ow_ref[...], preferred_element_type=jnp.float32)
                    + ob_ref[...])


def _tc5(g, xt, fg, bg, f1a, f1b, b1f, f2, b2f, ow, ob):
    return pl.pallas_call(
        _tc5_body,
        out_shape=jax.ShapeDtypeStruct((B, 1), jnp.float32),
    )(g, xt, fg, bg, f1a, f1b, b1f, f2, b2f, ow, ob)


def kernel(x, edge_index, batch, target, W1, att_src1, att_dst1, b1,
           W2, att_src2, att_dst2, b2, emb, conv_w, conv_b,
           fc_xt1_w, fc_xt1_b, fc_g1_w, fc_g1_b, fc1_w, fc1_b,
           fc2_w, fc2_b, out_w, out_b):
    srcf1 = edge_index[0].reshape(NS, NRB1, K)
    dst31 = edge_index[1].reshape(NS, NRB1, K)
    srcf2 = edge_index[0].reshape(NW, NRB2, K2)
    dst32 = edge_index[1].reshape(NW, NRB2, K2)
    w1h = W1.reshape(D, H, D).transpose(1, 0, 2)
    eye16 = jnp.eye(16, dtype=jnp.float32)[:H]
    asp = att_src1[:, :, None] * eye16[:, None, :]
    adp = att_dst1[:, :, None] * eye16[:, None, :]
    z16 = jnp.zeros((NPAD, 16), jnp.float32)
    zD = jnp.zeros((NPAD, D), jnp.float32)
    b1h = b1.reshape(H, D)
    w2h = W2.reshape(H, D, D)
    a2sp = att_src2[0][:, None] * jnp.eye(16, dtype=jnp.float32)[0][None, :]
    a2dp = att_dst2[0][:, None] * jnp.eye(16, dtype=jnp.float32)[0][None, :]
    batch2 = batch.reshape(N, 1)
    b2row = b2.reshape(1, D)

    xp, asrc, adst, ms1, md1 = _tc1(x, w1h, asp, adp)
    acc1, asum1, _w1x = _sc_gat1(srcf1, dst31, asrc, adst,
                           ms1.reshape(16), md1.reshape(16),
                           z16, zD, xp.reshape(H * NPAD, D))
    acc1 = acc1.reshape(H, NPAD, D)

    h2p, a2s, a2d, ms2, md2, st, en = _tc2(
        acc1, asum1, asrc, adst, xp, ms1, md1, batch2, b1h, w2h, a2sp, a2dp)

    acc2, asum2 = _sc_gat2(srcf2, dst32, a2s, a2d,
                           ms2.reshape(16), md2.reshape(16), z16, zD, h2p)

    h2 = _tc3(acc2.reshape(2, NPAD, D), asum2.reshape(2, NPAD, 16),
              a2s, a2d, h2p, ms2, md2, b2row)

    stpad = jnp.pad(st.reshape(B), (0, 16))
    enpad = jnp.pad(en.reshape(B), (0, 16))
    g = _sc_pool(h2.reshape(NPAD * D), stpad, enpad).reshape(B, D)

    t3 = target[:, :, None]
    emb32 = jnp.pad(emb, ((0, 32 - VOCAB), (0, 0)))
    wr2 = conv_w.transpose(2, 0, 1).reshape(B, SEQ)
    cb2 = conv_b.reshape(32, 1)
    fcw3 = fc_xt1_w.reshape(32, 121, D)
    xt = _tc4(t3, emb32, wr2, cb2, fcw3, fc_xt1_b.reshape(1, D)).reshape(B, D)

    return _tc5(g, xt, fc_g1_w, fc_g1_b.reshape(1, D),
                fc1_w[:D], fc1_w[D:], fc1_b.reshape(1, 1024),
                fc2_w, fc2_b.reshape(1, B), out_w, out_b.reshape(1, 1))
